# Initial kernel scaffold; baseline (speedup 1.0000x reference)
import functools

import jax
import jax.numpy as jnp
from jax import lax
from jax.experimental import pallas as pl
from jax.experimental.pallas import tpu as pltpu
from jax.experimental.pallas import tpu_sc as plsc

_N = 10000
_E = 320000
_DIN = 128
_H1 = 8
_C1 = 8
_D1 = 64
_A1 = 80
_NCLS = 40
_D2 = 48

_NC = 2
_NS = 16
_NW = _NC * _NS

_KCH = 128
_EPAD = 327680
_PERW = _EPAD // _NW
_NCHUNK = _PERW // _KCH

_RPT = _N // _NS
_RCH = 125
_NRCH = _RPT // _RCH

_BN = 1000
_BE = 2048


def _iota2(shape, dim):
    return lax.broadcasted_iota(jnp.int32, shape, dim)


def _headsum_mat():
    return (_iota2((_D1, _H1), 0) // _C1 == _iota2((_D1, _H1), 1)).astype(jnp.float32)


def _headexp_mat():
    return (_iota2((_H1, _D1), 1) // _C1 == _iota2((_H1, _D1), 0)).astype(jnp.float32)


def _emb_num_mat():
    return (_iota2((_D1, _A1), 0) == _iota2((_D1, _A1), 1)).astype(jnp.float32)


def _emb_den_mat():
    return (_iota2((_H1, _A1), 0) + _D1 == _iota2((_H1, _A1), 1)).astype(jnp.float32)


def _ext_num_mat():
    return (_iota2((_A1, _D1), 0) == _iota2((_A1, _D1), 1)).astype(jnp.float32)


def _ext_den_mat():
    return (_iota2((_A1, _H1), 0) == _iota2((_A1, _H1), 1) + _D1).astype(jnp.float32)


def _mask40():
    return (_iota2((1, _D2), 1) < _NCLS).astype(jnp.float32)


def _e40():
    return (_iota2((1, _D2), 1) == _NCLS).astype(jnp.float32)


def _leaky(s):
    return jnp.maximum(s, 0.2 * s)


def _node1_body(x_ref, wl_ref, wr_ref, attf_ref, xl_ref, xr_ref, self_ref):
    x = x_ref[...]
    xl = jnp.dot(x, wl_ref[...], preferred_element_type=jnp.float32)
    xr = jnp.dot(x, wr_ref[...], preferred_element_type=jnp.float32)
    w = _leaky(xl + xr) * attf_ref[...]
    alpha = jnp.dot(w, _headsum_mat(), preferred_element_type=jnp.float32)
    ex = jnp.exp(alpha)
    ex_e = jnp.dot(ex, _headexp_mat(), preferred_element_type=jnp.float32)
    num = xl * ex_e
    self_ref[...] = 0.5 * (
        jnp.dot(num, _emb_num_mat(), preferred_element_type=jnp.float32)
        + jnp.dot(ex, _emb_den_mat(), preferred_element_type=jnp.float32))
    xl_ref[...] = xl
    xr_ref[...] = xr


def _node1(x, wl, wr, attf):
    return pl.pallas_call(
        _node1_body,
        grid=(_N // _BN,),
        in_specs=[
            pl.BlockSpec((_BN, _DIN), lambda i: (i, 0)),
            pl.BlockSpec((_DIN, _D1), lambda i: (0, 0)),
            pl.BlockSpec((_DIN, _D1), lambda i: (0, 0)),
            pl.BlockSpec((1, _D1), lambda i: (0, 0)),
        ],
        out_specs=[
            pl.BlockSpec((_BN, _D1), lambda i: (i, 0)),
            pl.BlockSpec((_BN, _D1), lambda i: (i, 0)),
            pl.BlockSpec((_BN, _A1), lambda i: (i, 0)),
        ],
        out_shape=[
            jax.ShapeDtypeStruct((_N, _D1), jnp.float32),
            jax.ShapeDtypeStruct((_N, _D1), jnp.float32),
            jax.ShapeDtypeStruct((_N, _A1), jnp.float32),
        ],
    )(x, wl, wr, attf)


def _edge1_body(xl_ref, xr_ref, attf_ref, out_ref):
    pid = pl.program_id(0)
    xl = xl_ref[...]
    w = _leaky(xl + xr_ref[...]) * attf_ref[...]
    alpha = jnp.dot(w, _headsum_mat(), preferred_element_type=jnp.float32)
    ex = jnp.exp(alpha)
    ex_e = jnp.dot(ex, _headexp_mat(), preferred_element_type=jnp.float32)
    num = xl * ex_e
    contrib = (jnp.dot(num, _emb_num_mat(), preferred_element_type=jnp.float32)
               + jnp.dot(ex, _emb_den_mat(), preferred_element_type=jnp.float32))
    row = pid * _BE + _iota2((_BE, 1), 0)
    out_ref[...] = jnp.where(row < _E, contrib, 0.0)


def _edge1(xl_g, xr_g, attf):
    return pl.pallas_call(
        _edge1_body,
        grid=(_EPAD // _BE,),
        in_specs=[
            pl.BlockSpec((_BE, _D1), lambda i: (i, 0)),
            pl.BlockSpec((_BE, _D1), lambda i: (i, 0)),
            pl.BlockSpec((1, _D1), lambda i: (0, 0)),
        ],
        out_specs=pl.BlockSpec((_BE, _A1), lambda i: (i, 0)),
        out_shape=jax.ShapeDtypeStruct((_EPAD, _A1), jnp.float32),
    )(xl_g, xr_g, attf)


def _node2_body(p0_ref, p1_ref, b1_ref, wl_ref, wr_ref, att2_ref,
                xl_ref, xr_ref, self_ref):
    tot = p0_ref[...] + p1_ref[...]
    num = jnp.dot(tot, _ext_num_mat(), preferred_element_type=jnp.float32)
    den = jnp.dot(tot, _ext_den_mat(), preferred_element_type=jnp.float32)
    den_e = jnp.dot(den, _headexp_mat(), preferred_element_type=jnp.float32)
    h = num / (den_e + 1e-16) + b1_ref[...]
    h = jnp.where(h > 0, h, jnp.exp(jnp.minimum(h, 0.0)) - 1.0)
    xl2 = jnp.dot(h, wl_ref[...], preferred_element_type=jnp.float32)
    xr2 = jnp.dot(h, wr_ref[...], preferred_element_type=jnp.float32)
    w = _leaky(xl2 + xr2) * att2_ref[...]
    alpha = jnp.sum(w, axis=1, keepdims=True)
    ex = jnp.exp(alpha)
    self_ref[...] = 0.5 * (xl2 * ex * _mask40() + ex * _e40())
    xl_ref[...] = xl2
    xr_ref[...] = xr2


def _node2(p0, p1, b1r, wl2, wr2, att2f):
    return pl.pallas_call(
        _node2_body,
        grid=(_N // _BN,),
        in_specs=[
            pl.BlockSpec((_BN, _A1), lambda i: (i, 0)),
            pl.BlockSpec((_BN, _A1), lambda i: (i, 0)),
            pl.BlockSpec((1, _D1), lambda i: (0, 0)),
            pl.BlockSpec((_D1, _D2), lambda i: (0, 0)),
            pl.BlockSpec((_D1, _D2), lambda i: (0, 0)),
            pl.BlockSpec((1, _D2), lambda i: (0, 0)),
        ],
        out_specs=[
            pl.BlockSpec((_BN, _D2), lambda i: (i, 0)),
            pl.BlockSpec((_BN, _D2), lambda i: (i, 0)),
            pl.BlockSpec((_BN, _D2), lambda i: (i, 0)),
        ],
        out_shape=[
            jax.ShapeDtypeStruct((_N, _D2), jnp.float32),
            jax.ShapeDtypeStruct((_N, _D2), jnp.float32),
            jax.ShapeDtypeStruct((_N, _D2), jnp.float32),
        ],
    )(p0, p1, b1r, wl2, wr2, att2f)


def _edge2_body(xl_ref, xr_ref, att2_ref, out_ref):
    pid = pl.program_id(0)
    xl = xl_ref[...]
    w = _leaky(xl + xr_ref[...]) * att2_ref[...]
    alpha = jnp.sum(w, axis=1, keepdims=True)
    ex = jnp.exp(alpha)
    contrib = xl * ex * _mask40() + ex * _e40()
    row = pid * _BE + _iota2((_BE, 1), 0)
    out_ref[...] = jnp.where(row < _E, contrib, 0.0)


def _edge2(xl_g, xr_g, att2f):
    return pl.pallas_call(
        _edge2_body,
        grid=(_EPAD // _BE,),
        in_specs=[
            pl.BlockSpec((_BE, _D2), lambda i: (i, 0)),
            pl.BlockSpec((_BE, _D2), lambda i: (i, 0)),
            pl.BlockSpec((1, _D2), lambda i: (0, 0)),
        ],
        out_specs=pl.BlockSpec((_BE, _D2), lambda i: (i, 0)),
        out_shape=jax.ShapeDtypeStruct((_EPAD, _D2), jnp.float32),
    )(xl_g, xr_g, att2f)


def _final_body(q0_ref, q1_ref, b2_ref, out_ref):
    tot = q0_ref[...] + q1_ref[...]
    den = jnp.sum(tot * _e40(), axis=1, keepdims=True)
    logits = tot * _mask40() / (den + 1e-16) + b2_ref[...]
    z = jnp.where(_mask40() > 0, logits, -1e30)
    m = jnp.max(z, axis=1, keepdims=True)
    se = jnp.sum(jnp.exp(z - m), axis=1, keepdims=True)
    out48 = z - (jnp.log(se) + m)
    out_ref[...] = out48[:, :_NCLS]


def _final(q0, q1, b2p):
    return pl.pallas_call(
        _final_body,
        grid=(_N // _BN,),
        in_specs=[
            pl.BlockSpec((_BN, _D2), lambda i: (i, 0)),
            pl.BlockSpec((_BN, _D2), lambda i: (i, 0)),
            pl.BlockSpec((1, _D2), lambda i: (0, 0)),
        ],
        out_specs=pl.BlockSpec((_BN, _NCLS), lambda i: (i, 0)),
        out_shape=jax.ShapeDtypeStruct((_N, _NCLS), jnp.float32),
    )(q0, q1, b2p)


def _sc_mesh():
    return plsc.VectorSubcoreMesh(core_axis_name="c", subcore_axis_name="s")


def _make_sc_gather(d):
    def body(tl, tr, src2d, dst2d, outl, outr,
             idxs, idxd, rowsl, rowsr, seml, semr):
        wid = lax.axis_index("s") * _NC + lax.axis_index("c")
        pltpu.sync_copy(src2d.at[pl.ds(wid * _NCHUNK, _NCHUNK)], idxs)
        pltpu.sync_copy(dst2d.at[pl.ds(wid * _NCHUNK, _NCHUNK)], idxd)
        base = wid * _PERW

        def chunk(i, carry):
            off = base + i * _KCH
            cl = pltpu.async_copy(tl.at[idxs.at[i]], rowsl, seml)
            cr = pltpu.async_copy(tr.at[idxd.at[i]], rowsr, semr)
            cl.wait()
            pltpu.sync_copy(rowsl, outl.at[pl.ds(off, _KCH)])
            cr.wait()
            pltpu.sync_copy(rowsr, outr.at[pl.ds(off, _KCH)])
            return carry

        lax.fori_loop(0, _NCHUNK, chunk, 0)

    return functools.partial(
        pl.kernel, body,
        out_type=[
            jax.ShapeDtypeStruct((_EPAD, d), jnp.float32),
            jax.ShapeDtypeStruct((_EPAD, d), jnp.float32),
        ],
        mesh=_sc_mesh(),
        scratch_types=[
            pltpu.VMEM((_NCHUNK, _KCH), jnp.int32),
            pltpu.VMEM((_NCHUNK, _KCH), jnp.int32),
            pltpu.VMEM((_KCH, d), jnp.float32),
            pltpu.VMEM((_KCH, d), jnp.float32),
            pltpu.SemaphoreType.DMA,
            pltpu.SemaphoreType.DMA,
        ],
    )()


def _make_sc_scatter(d):
    def body(contrib, dst2d, init, out, idxd, rowsv, acc):
        cid = lax.axis_index("c")
        sid = lax.axis_index("s")
        wid = sid * _NC + cid
        r0 = sid * _RPT

        def initchunk(j, carry):
            rr = r0 + j * _RCH
            pltpu.sync_copy(init.at[pl.ds(rr, _RCH)], rowsv.at[pl.ds(0, _RCH)])
            pltpu.sync_copy(rowsv.at[pl.ds(0, _RCH)], acc.at[pl.ds(rr, _RCH)])
            return carry

        lax.fori_loop(0, _NRCH, initchunk, 0)
        plsc.subcore_barrier()

        pltpu.sync_copy(dst2d.at[pl.ds(wid * _NCHUNK, _NCHUNK)], idxd)
        base = wid * _PERW

        def chunk(i, carry):
            off = base + i * _KCH
            pltpu.sync_copy(contrib.at[pl.ds(off, _KCH)], rowsv)
            pltpu.sync_copy(rowsv, acc.at[idxd.at[i]], add=True)
            return carry

        lax.fori_loop(0, _NCHUNK, chunk, 0)
        plsc.subcore_barrier()

        def outchunk(j, carry):
            rr = r0 + j * _RCH
            pltpu.sync_copy(acc.at[pl.ds(rr, _RCH)], rowsv.at[pl.ds(0, _RCH)])
            pltpu.sync_copy(rowsv.at[pl.ds(0, _RCH)], out.at[cid, pl.ds(rr, _RCH)])
            return carry

        lax.fori_loop(0, _NRCH, outchunk, 0)

    return functools.partial(
        pl.kernel, body,
        out_type=jax.ShapeDtypeStruct((_NC, _N, d), jnp.float32),
        mesh=_sc_mesh(),
        scratch_types=[
            pltpu.VMEM((_NCHUNK, _KCH), jnp.int32),
            pltpu.VMEM((_KCH, d), jnp.float32),
            pltpu.VMEM_SHARED((_N, d), jnp.float32),
        ],
    )()


def _gather_rows(tl, tr, src2d, dst2d, d):
    return _make_sc_gather(d)(tl, tr, src2d, dst2d)


def _scatter_rows(contrib, dst2d, init, d):
    return _make_sc_scatter(d)(contrib, dst2d, init)


def kernel(x, edge_index, Wl1, Wr1, att1, b1, Wl2, Wr2, att2, b2):
    src = jnp.pad(edge_index[0], (0, _EPAD - _E)).reshape(-1, _KCH)
    dst = jnp.pad(edge_index[1], (0, _EPAD - _E)).reshape(-1, _KCH)

    attf1 = att1.reshape(1, _D1)
    b1r = b1.reshape(1, _D1)
    wl2p = jnp.pad(Wl2, ((0, 0), (0, _D2 - _NCLS)))
    wr2p = jnp.pad(Wr2, ((0, 0), (0, _D2 - _NCLS)))
    att2f = jnp.pad(att2.reshape(1, _NCLS), ((0, 0), (0, _D2 - _NCLS)))
    b2p = jnp.pad(b2.reshape(1, _NCLS), ((0, 0), (0, _D2 - _NCLS)))

    xl1, xr1, self1 = _node1(x, Wl1, Wr1, attf1)
    xl_g, xr_g = _gather_rows(xl1, xr1, src, dst, _D1)
    contrib1 = _edge1(xl_g, xr_g, attf1)
    p = _scatter_rows(contrib1, dst, self1, _A1)

    xl2, xr2, self2 = _node2(p[0], p[1], b1r, wl2p, wr2p, att2f)
    xl2_g, xr2_g = _gather_rows(xl2, xr2, src, dst, _D2)
    contrib2 = _edge2(xl2_g, xr2_g, att2f)
    q = _scatter_rows(contrib2, dst, self2, _D2)

    return _final(q[0], q[1], b2p)


# speedup vs baseline: 16.3988x; 16.3988x over previous
import functools

import jax
import jax.numpy as jnp
from jax import lax
from jax.experimental import pallas as pl
from jax.experimental.pallas import tpu as pltpu
from jax.experimental.pallas import tpu_sc as plsc

_N = 10000
_E = 320000
_DIN = 128
_H1 = 8
_C1 = 8
_D1 = 64
_A1 = 80
_NCLS = 40
_D2 = 48

_NC = 2
_NS = 16
_NW = _NC * _NS

_KCH = 128
_EPAD = 327680
_PERW = _EPAD // _NW
_NCHUNK = _PERW // _KCH

_NPAD = 10240
_RPT = _NPAD // _NS
_RCH = 128
_NRCH = _RPT // _RCH

_BN = 1000
_BE = 2048
_TW = 128


def _iota2(shape, dim):
    return lax.broadcasted_iota(jnp.int32, shape, dim)


def _headsum_mat():
    return (_iota2((_D1, _H1), 0) // _C1 == _iota2((_D1, _H1), 1)).astype(jnp.float32)


def _headexp_mat():
    return (_iota2((_H1, _D1), 1) // _C1 == _iota2((_H1, _D1), 0)).astype(jnp.float32)


def _emb_num_mat():
    return (_iota2((_D1, _A1), 0) == _iota2((_D1, _A1), 1)).astype(jnp.float32)


def _emb_den_mat():
    return (_iota2((_H1, _A1), 0) + _D1 == _iota2((_H1, _A1), 1)).astype(jnp.float32)


def _ext_num_mat():
    return (_iota2((_A1, _D1), 0) == _iota2((_A1, _D1), 1)).astype(jnp.float32)


def _ext_den_mat():
    return (_iota2((_A1, _H1), 0) == _iota2((_A1, _H1), 1) + _D1).astype(jnp.float32)


def _pack_mat(w, off):
    return (_iota2((w, _TW), 0) + off == _iota2((w, _TW), 1)).astype(jnp.float32)


def _unpack_mat(w, off):
    return (_iota2((_TW, w), 0) == _iota2((_TW, w), 1) + off).astype(jnp.float32)


def _mask40():
    return (_iota2((1, _D2), 1) < _NCLS).astype(jnp.float32)


def _e40():
    return (_iota2((1, _D2), 1) == _NCLS).astype(jnp.float32)


def _leaky(s):
    return jnp.maximum(s, 0.2 * s)


def _node1_body(x_ref, wl_ref, wr_ref, attf_ref, t_ref, self_ref):
    x = x_ref[...]
    xl = jnp.dot(x, wl_ref[...], preferred_element_type=jnp.float32)
    xr = jnp.dot(x, wr_ref[...], preferred_element_type=jnp.float32)
    w = _leaky(xl + xr) * attf_ref[...]
    alpha = jnp.dot(w, _headsum_mat(), preferred_element_type=jnp.float32)
    ex = jnp.exp(alpha)
    ex_e = jnp.dot(ex, _headexp_mat(), preferred_element_type=jnp.float32)
    num = xl * ex_e
    self_ref[...] = 0.5 * (
        jnp.dot(num, _emb_num_mat(), preferred_element_type=jnp.float32)
        + jnp.dot(ex, _emb_den_mat(), preferred_element_type=jnp.float32))
    t_ref[...] = (jnp.dot(xl, _pack_mat(_D1, 0), preferred_element_type=jnp.float32)
                  + jnp.dot(xr, _pack_mat(_D1, _D1), preferred_element_type=jnp.float32))


def _node1(x, wl, wr, attf):
    return pl.pallas_call(
        _node1_body,
        grid=(_N // _BN,),
        in_specs=[
            pl.BlockSpec((_BN, _DIN), lambda i: (i, 0)),
            pl.BlockSpec((_DIN, _D1), lambda i: (0, 0)),
            pl.BlockSpec((_DIN, _D1), lambda i: (0, 0)),
            pl.BlockSpec((1, _D1), lambda i: (0, 0)),
        ],
        out_specs=[
            pl.BlockSpec((_BN, _TW), lambda i: (i, 0)),
            pl.BlockSpec((_BN, _A1), lambda i: (i, 0)),
        ],
        out_shape=[
            jax.ShapeDtypeStruct((_N, _TW), jnp.float32),
            jax.ShapeDtypeStruct((_N, _A1), jnp.float32),
        ],
    )(x, wl, wr, attf)


def _edge1_body(gl_ref, gr_ref, attf_ref, out_ref):
    pid = pl.program_id(0)
    xl = jnp.dot(gl_ref[...], _unpack_mat(_D1, 0), preferred_element_type=jnp.float32)
    xr = jnp.dot(gr_ref[...], _unpack_mat(_D1, _D1), preferred_element_type=jnp.float32)
    w = _leaky(xl + xr) * attf_ref[...]
    alpha = jnp.dot(w, _headsum_mat(), preferred_element_type=jnp.float32)
    ex = jnp.exp(alpha)
    ex_e = jnp.dot(ex, _headexp_mat(), preferred_element_type=jnp.float32)
    num = xl * ex_e
    contrib = (jnp.dot(num, _emb_num_mat(), preferred_element_type=jnp.float32)
               + jnp.dot(ex, _emb_den_mat(), preferred_element_type=jnp.float32))
    row = pid * _BE + _iota2((_BE, 1), 0)
    out_ref[...] = jnp.where(row < _E, contrib, 0.0)


def _edge1(xl_g, xr_g, attf):
    return pl.pallas_call(
        _edge1_body,
        grid=(_EPAD // _BE,),
        in_specs=[
            pl.BlockSpec((_BE, _TW), lambda i: (i, 0)),
            pl.BlockSpec((_BE, _TW), lambda i: (i, 0)),
            pl.BlockSpec((1, _D1), lambda i: (0, 0)),
        ],
        out_specs=pl.BlockSpec((_BE, _A1), lambda i: (i, 0)),
        out_shape=jax.ShapeDtypeStruct((_EPAD, _A1), jnp.float32),
    )(xl_g, xr_g, attf)


def _node2_body(p0_ref, p1_ref, b1_ref, wl_ref, wr_ref, att2_ref,
                t_ref, self_ref):
    tot = p0_ref[...] + p1_ref[...]
    num = jnp.dot(tot, _ext_num_mat(), preferred_element_type=jnp.float32)
    den = jnp.dot(tot, _ext_den_mat(), preferred_element_type=jnp.float32)
    den_e = jnp.dot(den, _headexp_mat(), preferred_element_type=jnp.float32)
    h = num / (den_e + 1e-16) + b1_ref[...]
    h = jnp.where(h > 0, h, jnp.exp(jnp.minimum(h, 0.0)) - 1.0)
    xl2 = jnp.dot(h, wl_ref[...], preferred_element_type=jnp.float32)
    xr2 = jnp.dot(h, wr_ref[...], preferred_element_type=jnp.float32)
    w = _leaky(xl2 + xr2) * att2_ref[...]
    alpha = jnp.sum(w, axis=1, keepdims=True)
    ex = jnp.exp(alpha)
    self_ref[...] = 0.5 * (xl2 * ex * _mask40() + ex * _e40())
    t_ref[...] = (jnp.dot(xl2, _pack_mat(_D2, 0), preferred_element_type=jnp.float32)
                  + jnp.dot(xr2, _pack_mat(_D2, _D2), preferred_element_type=jnp.float32))


def _node2(p0, p1, b1r, wl2, wr2, att2f):
    return pl.pallas_call(
        _node2_body,
        grid=(_N // _BN,),
        in_specs=[
            pl.BlockSpec((_BN, _A1), lambda i: (i, 0)),
            pl.BlockSpec((_BN, _A1), lambda i: (i, 0)),
            pl.BlockSpec((1, _D1), lambda i: (0, 0)),
            pl.BlockSpec((_D1, _D2), lambda i: (0, 0)),
            pl.BlockSpec((_D1, _D2), lambda i: (0, 0)),
            pl.BlockSpec((1, _D2), lambda i: (0, 0)),
        ],
        out_specs=[
            pl.BlockSpec((_BN, _TW), lambda i: (i, 0)),
            pl.BlockSpec((_BN, _D2), lambda i: (i, 0)),
        ],
        out_shape=[
            jax.ShapeDtypeStruct((_N, _TW), jnp.float32),
            jax.ShapeDtypeStruct((_N, _D2), jnp.float32),
        ],
    )(p0, p1, b1r, wl2, wr2, att2f)


def _edge2_body(gl_ref, gr_ref, att2_ref, out_ref):
    pid = pl.program_id(0)
    xl = jnp.dot(gl_ref[...], _unpack_mat(_D2, 0), preferred_element_type=jnp.float32)
    xr = jnp.dot(gr_ref[...], _unpack_mat(_D2, _D2), preferred_element_type=jnp.float32)
    w = _leaky(xl + xr) * att2_ref[...]
    alpha = jnp.sum(w, axis=1, keepdims=True)
    ex = jnp.exp(alpha)
    contrib = xl * ex * _mask40() + ex * _e40()
    row = pid * _BE + _iota2((_BE, 1), 0)
    out_ref[...] = jnp.where(row < _E, contrib, 0.0)


def _edge2(xl_g, xr_g, att2f):
    return pl.pallas_call(
        _edge2_body,
        grid=(_EPAD // _BE,),
        in_specs=[
            pl.BlockSpec((_BE, _TW), lambda i: (i, 0)),
            pl.BlockSpec((_BE, _TW), lambda i: (i, 0)),
            pl.BlockSpec((1, _D2), lambda i: (0, 0)),
        ],
        out_specs=pl.BlockSpec((_BE, _D2), lambda i: (i, 0)),
        out_shape=jax.ShapeDtypeStruct((_EPAD, _D2), jnp.float32),
    )(xl_g, xr_g, att2f)


def _final_body(q0_ref, q1_ref, b2_ref, out_ref):
    tot = q0_ref[...] + q1_ref[...]
    den = jnp.sum(tot * _e40(), axis=1, keepdims=True)
    logits = tot * _mask40() / (den + 1e-16) + b2_ref[...]
    z = jnp.where(_mask40() > 0, logits, -1e30)
    m = jnp.max(z, axis=1, keepdims=True)
    se = jnp.sum(jnp.exp(z - m), axis=1, keepdims=True)
    out48 = z - (jnp.log(se) + m)
    out_ref[...] = out48[:, :_NCLS]


def _final(q0, q1, b2p):
    return pl.pallas_call(
        _final_body,
        grid=(_N // _BN,),
        in_specs=[
            pl.BlockSpec((_BN, _D2), lambda i: (i, 0)),
            pl.BlockSpec((_BN, _D2), lambda i: (i, 0)),
            pl.BlockSpec((1, _D2), lambda i: (0, 0)),
        ],
        out_specs=pl.BlockSpec((_BN, _NCLS), lambda i: (i, 0)),
        out_shape=jax.ShapeDtypeStruct((_N, _NCLS), jnp.float32),
    )(q0, q1, b2p)


def _sc_mesh():
    return plsc.VectorSubcoreMesh(core_axis_name="c", subcore_axis_name="s")


def _make_sc_gather():
    def body(t, src2d, dst2d, outl, outr,
             idxs, idxd, rowsl, rowsr, seml, semr):
        wid = lax.axis_index("s") * _NC + lax.axis_index("c")
        pltpu.sync_copy(src2d.at[pl.ds(wid * _NCHUNK, _NCHUNK)], idxs)
        pltpu.sync_copy(dst2d.at[pl.ds(wid * _NCHUNK, _NCHUNK)], idxd)
        base = wid * _PERW

        def chunk(i, carry):
            off = base + i * _KCH
            cl = pltpu.async_copy(t.at[idxs.at[i]], rowsl, seml)
            cr = pltpu.async_copy(t.at[idxd.at[i]], rowsr, semr)
            cl.wait()
            pltpu.sync_copy(rowsl, outl.at[pl.ds(off, _KCH)])
            cr.wait()
            pltpu.sync_copy(rowsr, outr.at[pl.ds(off, _KCH)])
            return carry

        lax.fori_loop(0, _NCHUNK, chunk, 0)

    return functools.partial(
        pl.kernel, body,
        out_type=[
            jax.ShapeDtypeStruct((_EPAD, _TW), jnp.float32),
            jax.ShapeDtypeStruct((_EPAD, _TW), jnp.float32),
        ],
        mesh=_sc_mesh(),
        scratch_types=[
            pltpu.VMEM((_NCHUNK, _KCH), jnp.int32),
            pltpu.VMEM((_NCHUNK, _KCH), jnp.int32),
            pltpu.VMEM((_KCH, _TW), jnp.float32),
            pltpu.VMEM((_KCH, _TW), jnp.float32),
            pltpu.SemaphoreType.DMA,
            pltpu.SemaphoreType.DMA,
        ],
    )()


def _make_sc_scatter(d):
    def body(contrib, dst2d, init, out, idxd, rowsv, acc):
        cid = lax.axis_index("c")
        sid = lax.axis_index("s")
        wid = sid * _NC + cid
        r0 = sid * _RPT

        def initchunk(j, carry):
            rr = r0 + j * _RCH
            pltpu.sync_copy(init.at[pl.ds(rr, _RCH)], rowsv)
            pltpu.sync_copy(rowsv, acc.at[pl.ds(rr, _RCH)])
            return carry

        lax.fori_loop(0, _NRCH, initchunk, 0)
        plsc.subcore_barrier()

        pltpu.sync_copy(dst2d.at[pl.ds(wid * _NCHUNK, _NCHUNK)], idxd)
        base = wid * _PERW

        def chunk(i, carry):
            off = base + i * _KCH
            pltpu.sync_copy(contrib.at[pl.ds(off, _KCH)], rowsv)
            pltpu.sync_copy(rowsv, acc.at[idxd.at[i]], add=True)
            return carry

        lax.fori_loop(0, _NCHUNK, chunk, 0)
        plsc.subcore_barrier()

        def outchunk(j, carry):
            rr = r0 + j * _RCH
            pltpu.sync_copy(acc.at[pl.ds(rr, _RCH)], rowsv)
            pltpu.sync_copy(rowsv, out.at[cid, pl.ds(rr, _RCH)])
            return carry

        lax.fori_loop(0, _NRCH, outchunk, 0)

    return functools.partial(
        pl.kernel, body,
        out_type=jax.ShapeDtypeStruct((_NC, _NPAD, d), jnp.float32),
        mesh=_sc_mesh(),
        scratch_types=[
            pltpu.VMEM((_NCHUNK, _KCH), jnp.int32),
            pltpu.VMEM((_KCH, d), jnp.float32),
            pltpu.VMEM_SHARED((_NPAD, d), jnp.float32),
        ],
    )()


def _gather_rows(t, src2d, dst2d):
    return _make_sc_gather()(t, src2d, dst2d)


def _scatter_rows(contrib, dst2d, init, d):
    return _make_sc_scatter(d)(contrib, dst2d, init)


def kernel(x, edge_index, Wl1, Wr1, att1, b1, Wl2, Wr2, att2, b2):
    src = jnp.pad(edge_index[0], (0, _EPAD - _E)).reshape(-1, _KCH)
    dst = jnp.pad(edge_index[1], (0, _EPAD - _E)).reshape(-1, _KCH)

    attf1 = att1.reshape(1, _D1)
    b1r = b1.reshape(1, _D1)
    wl2p = jnp.pad(Wl2, ((0, 0), (0, _D2 - _NCLS)))
    wr2p = jnp.pad(Wr2, ((0, 0), (0, _D2 - _NCLS)))
    att2f = jnp.pad(att2.reshape(1, _NCLS), ((0, 0), (0, _D2 - _NCLS)))
    b2p = jnp.pad(b2.reshape(1, _NCLS), ((0, 0), (0, _D2 - _NCLS)))

    t1, self1 = _node1(x, Wl1, Wr1, attf1)
    xl_g, xr_g = _gather_rows(t1, src, dst)
    contrib1 = _edge1(xl_g, xr_g, attf1)
    self1p = jnp.pad(self1, ((0, _NPAD - _N), (0, 0)))
    p = _scatter_rows(contrib1, dst, self1p, _A1)

    t2, self2 = _node2(p[0, :_N], p[1, :_N], b1r, wl2p, wr2p, att2f)
    xl2_g, xr2_g = _gather_rows(t2, src, dst)
    contrib2 = _edge2(xl2_g, xr2_g, att2f)
    self2p = jnp.pad(self2, ((0, _NPAD - _N), (0, 0)))
    q = _scatter_rows(contrib2, dst, self2p, _D2)

    return _final(q[0, :_N], q[1, :_N], b2p)


# speedup vs baseline: 37.9489x; 2.3141x over previous
import functools

import jax
import jax.numpy as jnp
from jax import lax
from jax.experimental import pallas as pl
from jax.experimental.pallas import tpu as pltpu
from jax.experimental.pallas import tpu_sc as plsc

_N = 10000
_E = 320000
_DIN = 128
_H1 = 8
_C1 = 8
_D1 = 64
_A1 = 80
_NCLS = 40
_D2 = 48

_NC = 2
_NS = 16
_NW = _NC * _NS

_KCH = 128
_EPAD = 327680
_PERW = _EPAD // _NW
_NCHUNK = _PERW // _KCH

_NPAD = 10240
_RPT = _NPAD // _NS
_RCH = 128
_NRCH = _RPT // _RCH

_BN = 1000
_BE = 2048
_TW = 128


def _iota2(shape, dim):
    return lax.broadcasted_iota(jnp.int32, shape, dim)


def _headsum_mat():
    return (_iota2((_D1, _H1), 0) // _C1 == _iota2((_D1, _H1), 1)).astype(jnp.float32)


def _headexp_mat():
    return (_iota2((_H1, _D1), 1) // _C1 == _iota2((_H1, _D1), 0)).astype(jnp.float32)


def _emb_num_mat():
    return (_iota2((_D1, _A1), 0) == _iota2((_D1, _A1), 1)).astype(jnp.float32)


def _emb_den_mat():
    return (_iota2((_H1, _A1), 0) + _D1 == _iota2((_H1, _A1), 1)).astype(jnp.float32)


def _ext_num_mat():
    return (_iota2((_A1, _D1), 0) == _iota2((_A1, _D1), 1)).astype(jnp.float32)


def _ext_den_mat():
    return (_iota2((_A1, _H1), 0) == _iota2((_A1, _H1), 1) + _D1).astype(jnp.float32)


def _pack_mat(w, off):
    return (_iota2((w, _TW), 0) + off == _iota2((w, _TW), 1)).astype(jnp.float32)


def _unpack_mat(w, off):
    return (_iota2((_TW, w), 0) == _iota2((_TW, w), 1) + off).astype(jnp.float32)


def _mask40():
    return (_iota2((1, _D2), 1) < _NCLS).astype(jnp.float32)


def _e40():
    return (_iota2((1, _D2), 1) == _NCLS).astype(jnp.float32)


def _leaky(s):
    return jnp.maximum(s, 0.2 * s)


def _node1_body(x_ref, wl_ref, wr_ref, attf_ref, t_ref, self_ref):
    x = x_ref[...]
    xl = jnp.dot(x, wl_ref[...], preferred_element_type=jnp.float32)
    xr = jnp.dot(x, wr_ref[...], preferred_element_type=jnp.float32)
    w = _leaky(xl + xr) * attf_ref[...]
    alpha = jnp.dot(w, _headsum_mat(), preferred_element_type=jnp.float32)
    ex = jnp.exp(alpha)
    ex_e = jnp.dot(ex, _headexp_mat(), preferred_element_type=jnp.float32)
    num = xl * ex_e
    self_ref[...] = 0.5 * (
        jnp.dot(num, _emb_num_mat(), preferred_element_type=jnp.float32)
        + jnp.dot(ex, _emb_den_mat(), preferred_element_type=jnp.float32))
    t_ref[...] = (jnp.dot(xl, _pack_mat(_D1, 0), preferred_element_type=jnp.float32)
                  + jnp.dot(xr, _pack_mat(_D1, _D1), preferred_element_type=jnp.float32))


def _node1(x, wl, wr, attf):
    return pl.pallas_call(
        _node1_body,
        grid=(_N // _BN,),
        in_specs=[
            pl.BlockSpec((_BN, _DIN), lambda i: (i, 0)),
            pl.BlockSpec((_DIN, _D1), lambda i: (0, 0)),
            pl.BlockSpec((_DIN, _D1), lambda i: (0, 0)),
            pl.BlockSpec((1, _D1), lambda i: (0, 0)),
        ],
        out_specs=[
            pl.BlockSpec((_BN, _TW), lambda i: (i, 0)),
            pl.BlockSpec((_BN, _A1), lambda i: (i, 0)),
        ],
        out_shape=[
            jax.ShapeDtypeStruct((_N, _TW), jnp.float32),
            jax.ShapeDtypeStruct((_N, _A1), jnp.float32),
        ],
    )(x, wl, wr, attf)


def _edge1_body(gl_ref, gr_ref, attf_ref, out_ref):
    pid = pl.program_id(0)
    xl = jnp.dot(gl_ref[...], _unpack_mat(_D1, 0), preferred_element_type=jnp.float32)
    xr = jnp.dot(gr_ref[...], _unpack_mat(_D1, _D1), preferred_element_type=jnp.float32)
    w = _leaky(xl + xr) * attf_ref[...]
    alpha = jnp.dot(w, _headsum_mat(), preferred_element_type=jnp.float32)
    ex = jnp.exp(alpha)
    ex_e = jnp.dot(ex, _headexp_mat(), preferred_element_type=jnp.float32)
    num = xl * ex_e
    contrib = (jnp.dot(num, _emb_num_mat(), preferred_element_type=jnp.float32)
               + jnp.dot(ex, _emb_den_mat(), preferred_element_type=jnp.float32))
    row = pid * _BE + _iota2((_BE, 1), 0)
    out_ref[...] = jnp.where(row < _E, contrib, 0.0)


def _edge1(xl_g, xr_g, attf):
    return pl.pallas_call(
        _edge1_body,
        grid=(_EPAD // _BE,),
        in_specs=[
            pl.BlockSpec((_BE, _TW), lambda i: (i, 0)),
            pl.BlockSpec((_BE, _TW), lambda i: (i, 0)),
            pl.BlockSpec((1, _D1), lambda i: (0, 0)),
        ],
        out_specs=pl.BlockSpec((_BE, _A1), lambda i: (i, 0)),
        out_shape=jax.ShapeDtypeStruct((_EPAD, _A1), jnp.float32),
    )(xl_g, xr_g, attf)


def _node2_body(p0_ref, p1_ref, b1_ref, wl_ref, wr_ref, att2_ref,
                t_ref, self_ref):
    tot = p0_ref[...] + p1_ref[...]
    num = jnp.dot(tot, _ext_num_mat(), preferred_element_type=jnp.float32)
    den = jnp.dot(tot, _ext_den_mat(), preferred_element_type=jnp.float32)
    den_e = jnp.dot(den, _headexp_mat(), preferred_element_type=jnp.float32)
    h = num / (den_e + 1e-16) + b1_ref[...]
    h = jnp.where(h > 0, h, jnp.exp(jnp.minimum(h, 0.0)) - 1.0)
    xl2 = jnp.dot(h, wl_ref[...], preferred_element_type=jnp.float32)
    xr2 = jnp.dot(h, wr_ref[...], preferred_element_type=jnp.float32)
    w = _leaky(xl2 + xr2) * att2_ref[...]
    alpha = jnp.sum(w, axis=1, keepdims=True)
    ex = jnp.exp(alpha)
    self_ref[...] = 0.5 * (xl2 * ex * _mask40() + ex * _e40())
    t_ref[...] = (jnp.dot(xl2, _pack_mat(_D2, 0), preferred_element_type=jnp.float32)
                  + jnp.dot(xr2, _pack_mat(_D2, _D2), preferred_element_type=jnp.float32))


def _node2(p0, p1, b1r, wl2, wr2, att2f):
    return pl.pallas_call(
        _node2_body,
        grid=(_N // _BN,),
        in_specs=[
            pl.BlockSpec((_BN, _A1), lambda i: (i, 0)),
            pl.BlockSpec((_BN, _A1), lambda i: (i, 0)),
            pl.BlockSpec((1, _D1), lambda i: (0, 0)),
            pl.BlockSpec((_D1, _D2), lambda i: (0, 0)),
            pl.BlockSpec((_D1, _D2), lambda i: (0, 0)),
            pl.BlockSpec((1, _D2), lambda i: (0, 0)),
        ],
        out_specs=[
            pl.BlockSpec((_BN, _TW), lambda i: (i, 0)),
            pl.BlockSpec((_BN, _D2), lambda i: (i, 0)),
        ],
        out_shape=[
            jax.ShapeDtypeStruct((_N, _TW), jnp.float32),
            jax.ShapeDtypeStruct((_N, _D2), jnp.float32),
        ],
    )(p0, p1, b1r, wl2, wr2, att2f)


def _edge2_body(gl_ref, gr_ref, att2_ref, out_ref):
    pid = pl.program_id(0)
    xl = jnp.dot(gl_ref[...], _unpack_mat(_D2, 0), preferred_element_type=jnp.float32)
    xr = jnp.dot(gr_ref[...], _unpack_mat(_D2, _D2), preferred_element_type=jnp.float32)
    w = _leaky(xl + xr) * att2_ref[...]
    alpha = jnp.sum(w, axis=1, keepdims=True)
    ex = jnp.exp(alpha)
    contrib = xl * ex * _mask40() + ex * _e40()
    row = pid * _BE + _iota2((_BE, 1), 0)
    out_ref[...] = jnp.where(row < _E, contrib, 0.0)


def _edge2(xl_g, xr_g, att2f):
    return pl.pallas_call(
        _edge2_body,
        grid=(_EPAD // _BE,),
        in_specs=[
            pl.BlockSpec((_BE, _TW), lambda i: (i, 0)),
            pl.BlockSpec((_BE, _TW), lambda i: (i, 0)),
            pl.BlockSpec((1, _D2), lambda i: (0, 0)),
        ],
        out_specs=pl.BlockSpec((_BE, _D2), lambda i: (i, 0)),
        out_shape=jax.ShapeDtypeStruct((_EPAD, _D2), jnp.float32),
    )(xl_g, xr_g, att2f)


def _final_body(q0_ref, q1_ref, b2_ref, out_ref):
    tot = q0_ref[...] + q1_ref[...]
    den = jnp.sum(tot * _e40(), axis=1, keepdims=True)
    logits = tot * _mask40() / (den + 1e-16) + b2_ref[...]
    z = jnp.where(_mask40() > 0, logits, -1e30)
    m = jnp.max(z, axis=1, keepdims=True)
    se = jnp.sum(jnp.exp(z - m), axis=1, keepdims=True)
    out48 = z - (jnp.log(se) + m)
    out_ref[...] = out48[:, :_NCLS]


def _final(q0, q1, b2p):
    return pl.pallas_call(
        _final_body,
        grid=(_N // _BN,),
        in_specs=[
            pl.BlockSpec((_BN, _D2), lambda i: (i, 0)),
            pl.BlockSpec((_BN, _D2), lambda i: (i, 0)),
            pl.BlockSpec((1, _D2), lambda i: (0, 0)),
        ],
        out_specs=pl.BlockSpec((_BN, _NCLS), lambda i: (i, 0)),
        out_shape=jax.ShapeDtypeStruct((_N, _NCLS), jnp.float32),
    )(q0, q1, b2p)


def _sc_mesh():
    return plsc.VectorSubcoreMesh(core_axis_name="c", subcore_axis_name="s")


def _make_sc_gather():
    def body(t, src2d, dst2d, outl, outr,
             idxs, idxd, rl0, rl1, rr0, rr1, sl0, sl1, sr0, sr1):
        wid = lax.axis_index("s") * _NC + lax.axis_index("c")
        pltpu.sync_copy(src2d.at[pl.ds(wid * _NCHUNK, _NCHUNK)], idxs)
        pltpu.sync_copy(dst2d.at[pl.ds(wid * _NCHUNK, _NCHUNK)], idxd)
        base = wid * _PERW
        bufs = ((rl0, sl0, rr0, sr0), (rl1, sl1, rr1, sr1))

        for b in range(2):
            rl, sl, rr, sr = bufs[b]
            pltpu.async_copy(t.at[idxs.at[b]], rl, sl)
            pltpu.async_copy(t.at[idxd.at[b]], rr, sr)

        def pair(g, carry):
            i = 2 * g
            for b in range(2):
                rl, sl, rr, sr = bufs[b]
                off = base + (i + b) * _KCH
                pltpu.make_async_copy(t.at[idxs.at[i + b]], rl, sl).wait()
                pltpu.sync_copy(rl, outl.at[pl.ds(off, _KCH)])
                pltpu.async_copy(t.at[idxs.at[i + b + 2]], rl, sl)
                pltpu.make_async_copy(t.at[idxd.at[i + b]], rr, sr).wait()
                pltpu.sync_copy(rr, outr.at[pl.ds(off, _KCH)])
                pltpu.async_copy(t.at[idxd.at[i + b + 2]], rr, sr)
            return carry

        lax.fori_loop(0, _NCHUNK // 2 - 1, pair, 0)

        i = _NCHUNK - 2
        for b in range(2):
            rl, sl, rr, sr = bufs[b]
            off = base + (i + b) * _KCH
            pltpu.make_async_copy(t.at[idxs.at[i + b]], rl, sl).wait()
            pltpu.sync_copy(rl, outl.at[pl.ds(off, _KCH)])
            pltpu.make_async_copy(t.at[idxd.at[i + b]], rr, sr).wait()
            pltpu.sync_copy(rr, outr.at[pl.ds(off, _KCH)])

    return functools.partial(
        pl.kernel, body,
        out_type=[
            jax.ShapeDtypeStruct((_EPAD, _TW), jnp.float32),
            jax.ShapeDtypeStruct((_EPAD, _TW), jnp.float32),
        ],
        mesh=_sc_mesh(),
        scratch_types=[
            pltpu.VMEM((_NCHUNK, _KCH), jnp.int32),
            pltpu.VMEM((_NCHUNK, _KCH), jnp.int32),
            pltpu.VMEM((_KCH, _TW), jnp.float32),
            pltpu.VMEM((_KCH, _TW), jnp.float32),
            pltpu.VMEM((_KCH, _TW), jnp.float32),
            pltpu.VMEM((_KCH, _TW), jnp.float32),
            pltpu.SemaphoreType.DMA,
            pltpu.SemaphoreType.DMA,
            pltpu.SemaphoreType.DMA,
            pltpu.SemaphoreType.DMA,
        ],
    )()


def _make_sc_scatter(d):
    def body(contrib, dst2d, init, out, idxd, rowsv, acc):
        cid = lax.axis_index("c")
        sid = lax.axis_index("s")
        wid = sid * _NC + cid
        r0 = sid * _RPT

        def initchunk(j, carry):
            rr = r0 + j * _RCH
            pltpu.sync_copy(init.at[pl.ds(rr, _RCH)], rowsv)
            pltpu.sync_copy(rowsv, acc.at[pl.ds(rr, _RCH)])
            return carry

        lax.fori_loop(0, _NRCH, initchunk, 0)
        plsc.subcore_barrier()

        pltpu.sync_copy(dst2d.at[pl.ds(wid * _NCHUNK, _NCHUNK)], idxd)
        base = wid * _PERW

        def chunk(i, carry):
            off = base + i * _KCH
            pltpu.sync_copy(contrib.at[pl.ds(off, _KCH)], rowsv)
            pltpu.sync_copy(rowsv, acc.at[idxd.at[i]], add=True)
            return carry

        lax.fori_loop(0, _NCHUNK, chunk, 0)
        plsc.subcore_barrier()

        def outchunk(j, carry):
            rr = r0 + j * _RCH
            pltpu.sync_copy(acc.at[pl.ds(rr, _RCH)], rowsv)
            pltpu.sync_copy(rowsv, out.at[cid, pl.ds(rr, _RCH)])
            return carry

        lax.fori_loop(0, _NRCH, outchunk, 0)

    return functools.partial(
        pl.kernel, body,
        out_type=jax.ShapeDtypeStruct((_NC, _NPAD, d), jnp.float32),
        mesh=_sc_mesh(),
        scratch_types=[
            pltpu.VMEM((_NCHUNK, _KCH), jnp.int32),
            pltpu.VMEM((_KCH, d), jnp.float32),
            pltpu.VMEM_SHARED((_NPAD, d), jnp.float32),
        ],
    )()


def _gather_rows(t, src2d, dst2d):
    return _make_sc_gather()(t, src2d, dst2d)


def _scatter_rows(contrib, dst2d, init, d):
    return _make_sc_scatter(d)(contrib, dst2d, init)


def kernel(x, edge_index, Wl1, Wr1, att1, b1, Wl2, Wr2, att2, b2):
    padidx = (jnp.arange(_EPAD - _E, dtype=jnp.int32) * 16) % _N
    src = jnp.concatenate([edge_index[0], padidx]).reshape(-1, _KCH)
    dst = jnp.concatenate([edge_index[1], padidx]).reshape(-1, _KCH)

    attf1 = att1.reshape(1, _D1)
    b1r = b1.reshape(1, _D1)
    wl2p = jnp.pad(Wl2, ((0, 0), (0, _D2 - _NCLS)))
    wr2p = jnp.pad(Wr2, ((0, 0), (0, _D2 - _NCLS)))
    att2f = jnp.pad(att2.reshape(1, _NCLS), ((0, 0), (0, _D2 - _NCLS)))
    b2p = jnp.pad(b2.reshape(1, _NCLS), ((0, 0), (0, _D2 - _NCLS)))

    t1, self1 = _node1(x, Wl1, Wr1, attf1)
    xl_g, xr_g = _gather_rows(t1, src, dst)
    contrib1 = _edge1(xl_g, xr_g, attf1)
    self1p = jnp.pad(self1, ((0, _NPAD - _N), (0, 0)))
    p = _scatter_rows(contrib1, dst, self1p, _A1)

    t2, self2 = _node2(p[0, :_N], p[1, :_N], b1r, wl2p, wr2p, att2f)
    xl2_g, xr2_g = _gather_rows(t2, src, dst)
    contrib2 = _edge2(xl2_g, xr2_g, att2f)
    self2p = jnp.pad(self2, ((0, _NPAD - _N), (0, 0)))
    q = _scatter_rows(contrib2, dst, self2p, _D2)

    return _final(q[0, :_N], q[1, :_N], b2p)


# speedup vs baseline: 37.9813x; 1.0009x over previous
import functools

import jax
import jax.numpy as jnp
from jax import lax
from jax.experimental import pallas as pl
from jax.experimental.pallas import tpu as pltpu
from jax.experimental.pallas import tpu_sc as plsc

_N = 10000
_E = 320000
_DIN = 128
_H1 = 8
_C1 = 8
_D1 = 64
_A1 = 80
_NCLS = 40
_D2 = 48

_NC = 2
_NS = 16
_NW = _NC * _NS

_KCH = 128
_EPAD = 327680
_PERW = _EPAD // _NW
_NCHUNK = _PERW // _KCH

_NPAD = 10240
_RPT = _NPAD // _NS
_RCH = 128
_NRCH = _RPT // _RCH

_BN = 1000
_BE = 2048
_TW = 128


def _iota2(shape, dim):
    return lax.broadcasted_iota(jnp.int32, shape, dim)


def _headsum_mat():
    return (_iota2((_D1, _H1), 0) // _C1 == _iota2((_D1, _H1), 1)).astype(jnp.float32)


def _headexp_mat():
    return (_iota2((_H1, _D1), 1) // _C1 == _iota2((_H1, _D1), 0)).astype(jnp.float32)


def _emb_num_mat():
    return (_iota2((_D1, _A1), 0) == _iota2((_D1, _A1), 1)).astype(jnp.float32)


def _emb_den_mat():
    return (_iota2((_H1, _A1), 0) + _D1 == _iota2((_H1, _A1), 1)).astype(jnp.float32)


def _ext_num_mat():
    return (_iota2((_A1, _D1), 0) == _iota2((_A1, _D1), 1)).astype(jnp.float32)


def _ext_den_mat():
    return (_iota2((_A1, _H1), 0) == _iota2((_A1, _H1), 1) + _D1).astype(jnp.float32)


def _pack_mat(w, off):
    return (_iota2((w, _TW), 0) + off == _iota2((w, _TW), 1)).astype(jnp.float32)


def _unpack_mat(w, off):
    return (_iota2((_TW, w), 0) == _iota2((_TW, w), 1) + off).astype(jnp.float32)


def _mask40():
    return (_iota2((1, _D2), 1) < _NCLS).astype(jnp.float32)


def _e40():
    return (_iota2((1, _D2), 1) == _NCLS).astype(jnp.float32)


def _leaky(s):
    return jnp.maximum(s, 0.2 * s)


def _node1_body(x_ref, wl_ref, wr_ref, attf_ref, t_ref, self_ref):
    x = x_ref[...]
    xl = jnp.dot(x, wl_ref[...], preferred_element_type=jnp.float32)
    xr = jnp.dot(x, wr_ref[...], preferred_element_type=jnp.float32)
    w = _leaky(xl + xr) * attf_ref[...]
    alpha = jnp.dot(w, _headsum_mat(), preferred_element_type=jnp.float32)
    ex = jnp.exp(alpha)
    ex_e = jnp.dot(ex, _headexp_mat(), preferred_element_type=jnp.float32)
    num = xl * ex_e
    self_ref[...] = 0.5 * (
        jnp.dot(num, _emb_num_mat(), preferred_element_type=jnp.float32)
        + jnp.dot(ex, _emb_den_mat(), preferred_element_type=jnp.float32))
    t_ref[...] = (jnp.dot(xl, _pack_mat(_D1, 0), preferred_element_type=jnp.float32)
                  + jnp.dot(xr, _pack_mat(_D1, _D1), preferred_element_type=jnp.float32))


def _node1(x, wl, wr, attf):
    return pl.pallas_call(
        _node1_body,
        grid=(_N // _BN,),
        in_specs=[
            pl.BlockSpec((_BN, _DIN), lambda i: (i, 0)),
            pl.BlockSpec((_DIN, _D1), lambda i: (0, 0)),
            pl.BlockSpec((_DIN, _D1), lambda i: (0, 0)),
            pl.BlockSpec((1, _D1), lambda i: (0, 0)),
        ],
        out_specs=[
            pl.BlockSpec((_BN, _TW), lambda i: (i, 0)),
            pl.BlockSpec((_BN, _A1), lambda i: (i, 0)),
        ],
        out_shape=[
            jax.ShapeDtypeStruct((_N, _TW), jnp.float32),
            jax.ShapeDtypeStruct((_N, _A1), jnp.float32),
        ],
    )(x, wl, wr, attf)


def _edge1_body(gl_ref, gr_ref, attf_ref, out_ref):
    pid = pl.program_id(0)
    xl = jnp.dot(gl_ref[...], _unpack_mat(_D1, 0), preferred_element_type=jnp.float32)
    xr = jnp.dot(gr_ref[...], _unpack_mat(_D1, _D1), preferred_element_type=jnp.float32)
    w = _leaky(xl + xr) * attf_ref[...]
    alpha = jnp.dot(w, _headsum_mat(), preferred_element_type=jnp.float32)
    ex = jnp.exp(alpha)
    ex_e = jnp.dot(ex, _headexp_mat(), preferred_element_type=jnp.float32)
    num = xl * ex_e
    contrib = (jnp.dot(num, _emb_num_mat(), preferred_element_type=jnp.float32)
               + jnp.dot(ex, _emb_den_mat(), preferred_element_type=jnp.float32))
    row = pid * _BE + _iota2((_BE, 1), 0)
    out_ref[...] = jnp.where(row < _E, contrib, 0.0)


def _edge1(xl_g, xr_g, attf):
    return pl.pallas_call(
        _edge1_body,
        grid=(_EPAD // _BE,),
        in_specs=[
            pl.BlockSpec((_BE, _TW), lambda i: (i, 0)),
            pl.BlockSpec((_BE, _TW), lambda i: (i, 0)),
            pl.BlockSpec((1, _D1), lambda i: (0, 0)),
        ],
        out_specs=pl.BlockSpec((_BE, _A1), lambda i: (i, 0)),
        out_shape=jax.ShapeDtypeStruct((_EPAD, _A1), jnp.float32),
    )(xl_g, xr_g, attf)


def _node2_body(p0_ref, p1_ref, b1_ref, wl_ref, wr_ref, att2_ref,
                t_ref, self_ref):
    tot = p0_ref[...] + p1_ref[...]
    num = jnp.dot(tot, _ext_num_mat(), preferred_element_type=jnp.float32)
    den = jnp.dot(tot, _ext_den_mat(), preferred_element_type=jnp.float32)
    den_e = jnp.dot(den, _headexp_mat(), preferred_element_type=jnp.float32)
    h = num / (den_e + 1e-16) + b1_ref[...]
    h = jnp.where(h > 0, h, jnp.exp(jnp.minimum(h, 0.0)) - 1.0)
    xl2 = jnp.dot(h, wl_ref[...], preferred_element_type=jnp.float32)
    xr2 = jnp.dot(h, wr_ref[...], preferred_element_type=jnp.float32)
    w = _leaky(xl2 + xr2) * att2_ref[...]
    alpha = jnp.sum(w, axis=1, keepdims=True)
    ex = jnp.exp(alpha)
    self_ref[...] = 0.5 * (xl2 * ex * _mask40() + ex * _e40())
    t_ref[...] = (jnp.dot(xl2, _pack_mat(_D2, 0), preferred_element_type=jnp.float32)
                  + jnp.dot(xr2, _pack_mat(_D2, _D1), preferred_element_type=jnp.float32))


def _node2(p0, p1, b1r, wl2, wr2, att2f):
    return pl.pallas_call(
        _node2_body,
        grid=(_N // _BN,),
        in_specs=[
            pl.BlockSpec((_BN, _A1), lambda i: (i, 0)),
            pl.BlockSpec((_BN, _A1), lambda i: (i, 0)),
            pl.BlockSpec((1, _D1), lambda i: (0, 0)),
            pl.BlockSpec((_D1, _D2), lambda i: (0, 0)),
            pl.BlockSpec((_D1, _D2), lambda i: (0, 0)),
            pl.BlockSpec((1, _D2), lambda i: (0, 0)),
        ],
        out_specs=[
            pl.BlockSpec((_BN, _TW), lambda i: (i, 0)),
            pl.BlockSpec((_BN, _D2), lambda i: (i, 0)),
        ],
        out_shape=[
            jax.ShapeDtypeStruct((_N, _TW), jnp.float32),
            jax.ShapeDtypeStruct((_N, _D2), jnp.float32),
        ],
    )(p0, p1, b1r, wl2, wr2, att2f)


def _edge2_body(gl_ref, gr_ref, att2_ref, out_ref):
    pid = pl.program_id(0)
    xl = jnp.dot(gl_ref[...], _unpack_mat(_D2, 0), preferred_element_type=jnp.float32)
    xr = jnp.dot(gr_ref[...], _unpack_mat(_D2, _D1), preferred_element_type=jnp.float32)
    w = _leaky(xl + xr) * att2_ref[...]
    alpha = jnp.sum(w, axis=1, keepdims=True)
    ex = jnp.exp(alpha)
    contrib = xl * ex * _mask40() + ex * _e40()
    row = pid * _BE + _iota2((_BE, 1), 0)
    out_ref[...] = jnp.where(row < _E, contrib, 0.0)


def _edge2(xl_g, xr_g, att2f):
    return pl.pallas_call(
        _edge2_body,
        grid=(_EPAD // _BE,),
        in_specs=[
            pl.BlockSpec((_BE, _TW), lambda i: (i, 0)),
            pl.BlockSpec((_BE, _TW), lambda i: (i, 0)),
            pl.BlockSpec((1, _D2), lambda i: (0, 0)),
        ],
        out_specs=pl.BlockSpec((_BE, _D2), lambda i: (i, 0)),
        out_shape=jax.ShapeDtypeStruct((_EPAD, _D2), jnp.float32),
    )(xl_g, xr_g, att2f)


def _final_body(q0_ref, q1_ref, b2_ref, out_ref):
    tot = q0_ref[...] + q1_ref[...]
    den = jnp.sum(tot * _e40(), axis=1, keepdims=True)
    logits = tot * _mask40() / (den + 1e-16) + b2_ref[...]
    z = jnp.where(_mask40() > 0, logits, -1e30)
    m = jnp.max(z, axis=1, keepdims=True)
    se = jnp.sum(jnp.exp(z - m), axis=1, keepdims=True)
    out48 = z - (jnp.log(se) + m)
    out_ref[...] = out48[:, :_NCLS]


def _final(q0, q1, b2p):
    return pl.pallas_call(
        _final_body,
        grid=(_N // _BN,),
        in_specs=[
            pl.BlockSpec((_BN, _D2), lambda i: (i, 0)),
            pl.BlockSpec((_BN, _D2), lambda i: (i, 0)),
            pl.BlockSpec((1, _D2), lambda i: (0, 0)),
        ],
        out_specs=pl.BlockSpec((_BN, _NCLS), lambda i: (i, 0)),
        out_shape=jax.ShapeDtypeStruct((_N, _NCLS), jnp.float32),
    )(q0, q1, b2p)


def _sc_mesh():
    return plsc.VectorSubcoreMesh(core_axis_name="c", subcore_axis_name="s")


def _make_sc_gather():
    def body(t, src2d, dst2d, outl, outr,
             idxs, idxd, rl0, rl1, rr0, rr1, sl0, sl1, sr0, sr1):
        wid = lax.axis_index("s") * _NC + lax.axis_index("c")
        pltpu.sync_copy(src2d.at[pl.ds(wid * _NCHUNK, _NCHUNK)], idxs)
        pltpu.sync_copy(dst2d.at[pl.ds(wid * _NCHUNK, _NCHUNK)], idxd)
        base = wid * _PERW
        bufs = ((rl0, sl0, rr0, sr0), (rl1, sl1, rr1, sr1))

        for b in range(2):
            rl, sl, rr, sr = bufs[b]
            pltpu.async_copy(t.at[idxs.at[b]], rl, sl)
            pltpu.async_copy(t.at[idxd.at[b]], rr, sr)

        def pair(g, carry):
            i = 2 * g
            for b in range(2):
                rl, sl, rr, sr = bufs[b]
                off = base + (i + b) * _KCH
                pltpu.make_async_copy(t.at[idxs.at[i + b]], rl, sl).wait()
                pltpu.sync_copy(rl, outl.at[pl.ds(off, _KCH)])
                pltpu.async_copy(t.at[idxs.at[i + b + 2]], rl, sl)
                pltpu.make_async_copy(t.at[idxd.at[i + b]], rr, sr).wait()
                pltpu.sync_copy(rr, outr.at[pl.ds(off, _KCH)])
                pltpu.async_copy(t.at[idxd.at[i + b + 2]], rr, sr)
            return carry

        lax.fori_loop(0, _NCHUNK // 2 - 1, pair, 0)

        i = _NCHUNK - 2
        for b in range(2):
            rl, sl, rr, sr = bufs[b]
            off = base + (i + b) * _KCH
            pltpu.make_async_copy(t.at[idxs.at[i + b]], rl, sl).wait()
            pltpu.sync_copy(rl, outl.at[pl.ds(off, _KCH)])
            pltpu.make_async_copy(t.at[idxd.at[i + b]], rr, sr).wait()
            pltpu.sync_copy(rr, outr.at[pl.ds(off, _KCH)])

    return functools.partial(
        pl.kernel, body,
        out_type=[
            jax.ShapeDtypeStruct((_EPAD, _TW), jnp.float32),
            jax.ShapeDtypeStruct((_EPAD, _TW), jnp.float32),
        ],
        mesh=_sc_mesh(),
        scratch_types=[
            pltpu.VMEM((_NCHUNK, _KCH), jnp.int32),
            pltpu.VMEM((_NCHUNK, _KCH), jnp.int32),
            pltpu.VMEM((_KCH, _TW), jnp.float32),
            pltpu.VMEM((_KCH, _TW), jnp.float32),
            pltpu.VMEM((_KCH, _TW), jnp.float32),
            pltpu.VMEM((_KCH, _TW), jnp.float32),
            pltpu.SemaphoreType.DMA,
            pltpu.SemaphoreType.DMA,
            pltpu.SemaphoreType.DMA,
            pltpu.SemaphoreType.DMA,
        ],
    )()


def _make_sc_scatter(d):
    def body(contrib, dst2d, init, out, idxd, rowsv, acc):
        cid = lax.axis_index("c")
        sid = lax.axis_index("s")
        wid = sid * _NC + cid
        r0 = sid * _RPT

        def initchunk(j, carry):
            rr = r0 + j * _RCH
            pltpu.sync_copy(init.at[pl.ds(rr, _RCH)], rowsv)
            pltpu.sync_copy(rowsv, acc.at[pl.ds(rr, _RCH)])
            return carry

        lax.fori_loop(0, _NRCH, initchunk, 0)
        plsc.subcore_barrier()

        pltpu.sync_copy(dst2d.at[pl.ds(wid * _NCHUNK, _NCHUNK)], idxd)
        base = wid * _PERW

        def chunk(i, carry):
            off = base + i * _KCH
            pltpu.sync_copy(contrib.at[pl.ds(off, _KCH)], rowsv)
            pltpu.sync_copy(rowsv, acc.at[idxd.at[i]], add=True)
            return carry

        lax.fori_loop(0, _NCHUNK, chunk, 0)
        plsc.subcore_barrier()

        def outchunk(j, carry):
            rr = r0 + j * _RCH
            pltpu.sync_copy(acc.at[pl.ds(rr, _RCH)], rowsv)
            pltpu.sync_copy(rowsv, out.at[cid, pl.ds(rr, _RCH)])
            return carry

        lax.fori_loop(0, _NRCH, outchunk, 0)

    return functools.partial(
        pl.kernel, body,
        out_type=jax.ShapeDtypeStruct((_NC, _NPAD, d), jnp.float32),
        mesh=_sc_mesh(),
        scratch_types=[
            pltpu.VMEM((_NCHUNK, _KCH), jnp.int32),
            pltpu.VMEM((_KCH, d), jnp.float32),
            pltpu.VMEM_SHARED((_NPAD, d), jnp.float32),
        ],
    )()


def _gather_rows(t, src2d, dst2d):
    return _make_sc_gather()(t, src2d, dst2d)


def _scatter_rows(contrib, dst2d, init, d):
    return _make_sc_scatter(d)(contrib, dst2d, init)


def kernel(x, edge_index, Wl1, Wr1, att1, b1, Wl2, Wr2, att2, b2):
    padidx = (jnp.arange(_EPAD - _E, dtype=jnp.int32) * 16) % _N
    src = jnp.concatenate([edge_index[0], padidx]).reshape(-1, _KCH)
    dst = jnp.concatenate([edge_index[1], padidx]).reshape(-1, _KCH)

    attf1 = att1.reshape(1, _D1)
    b1r = b1.reshape(1, _D1)
    wl2p = jnp.pad(Wl2, ((0, 0), (0, _D2 - _NCLS)))
    wr2p = jnp.pad(Wr2, ((0, 0), (0, _D2 - _NCLS)))
    att2f = jnp.pad(att2.reshape(1, _NCLS), ((0, 0), (0, _D2 - _NCLS)))
    b2p = jnp.pad(b2.reshape(1, _NCLS), ((0, 0), (0, _D2 - _NCLS)))

    t1, self1 = _node1(x, Wl1, Wr1, attf1)
    xl_g, xr_g = _gather_rows(t1, src, dst)
    contrib1 = _edge1(xl_g, xr_g, attf1)
    self1p = jnp.pad(self1, ((0, _NPAD - _N), (0, 0)))
    p = _scatter_rows(contrib1, dst, self1p, _A1)

    t2, self2 = _node2(p[0, :_N], p[1, :_N], b1r, wl2p, wr2p, att2f)
    xl2_g, xr2_g = _gather_rows(t2, src, dst)
    contrib2 = _edge2(xl2_g, xr2_g, att2f)
    self2p = jnp.pad(self2, ((0, _NPAD - _N), (0, 0)))
    q = _scatter_rows(contrib2, dst, self2p, _D2)

    return _final(q[0, :_N], q[1, :_N], b2p)
